# Initial kernel scaffold; baseline (speedup 1.0000x reference)
import jax
import jax.numpy as jnp
from jax.experimental import pallas as pl


def kernel(x, edge_index, edge_vals, W, b):
    raise NotImplementedError("write your pallas kernel here")


# speedup vs baseline: 5.6233x; 5.6233x over previous
import dataclasses
import functools

import jax
import jax.numpy as jnp
from jax import lax
from jax.experimental import pallas as pl
from jax.experimental.pallas import tpu as pltpu
from jax.experimental.pallas import tpu_sc as plsc

N = 10000
D = 128
NC = 2
NS = 16
NT = NC * NS
CH = 128
NCH = 80
EPAD = NT * NCH * CH
NPAD = 10240
ROWS_PER_TILE = NPAD // NS
CPY = 128


def _sc_segment_sum(x, col3, row3, val3):
    mesh = plsc.VectorSubcoreMesh(core_axis_name="c", subcore_axis_name="s")
    cp = pltpu.CompilerParams()
    if "needs_layout_passes" in pltpu.CompilerParams.__dataclass_fields__:
        cp = dataclasses.replace(cp, needs_layout_passes=False)

    @functools.partial(
        pl.kernel,
        compiler_params=cp,
        out_type=jax.ShapeDtypeStruct((NC, NPAD, D), jnp.float32),
        mesh=mesh,
        scratch_types=[
            pltpu.VMEM((NCH, CH), jnp.int32),
            pltpu.VMEM((NCH, CH), jnp.int32),
            pltpu.VMEM((NCH, CH), jnp.float32),
            pltpu.VMEM((CH, D), jnp.float32),
            pltpu.VMEM_SHARED((NPAD, D), jnp.float32),
        ],
    )
    def sc_kernel(x_hbm, col_hbm, row_hbm, val_hbm, out_hbm,
                  col_v, row_v, val_v, buf, agg):
        c = lax.axis_index("c")
        s = lax.axis_index("s")
        wid = c * NS + s

        zero = jnp.zeros((16,), jnp.float32)

        @pl.loop(0, CH)
        def _zero_buf(r):
            for k in range(D // 16):
                buf[r, pl.ds(k * 16, 16)] = zero

        for i in range(ROWS_PER_TILE // CPY):
            pltpu.sync_copy(buf.at[pl.ds(0, CPY)],
                            agg.at[pl.ds(s * ROWS_PER_TILE + i * CPY, CPY)])

        pltpu.sync_copy(col_hbm.at[wid], col_v)
        pltpu.sync_copy(row_hbm.at[wid], row_v)
        pltpu.sync_copy(val_hbm.at[wid], val_v)
        plsc.subcore_barrier()

        @pl.loop(0, NCH)
        def _chunk(j):
            pltpu.sync_copy(x_hbm.at[col_v.at[j]], buf)
            j16 = jnp.full((16,), j, jnp.int32)

            @pl.loop(0, CH)
            def _scale(e):
                e16 = jnp.full((16,), e, jnp.int32)
                v = plsc.load_gather(val_v, [j16, e16])
                for k in range(D // 16):
                    sl = pl.ds(k * 16, 16)
                    buf[e, sl] = buf[e, sl] * v

            pltpu.sync_copy(buf, agg.at[row_v.at[j]], add=True)

        plsc.subcore_barrier()
        for i in range(ROWS_PER_TILE // CPY):
            st = s * ROWS_PER_TILE + i * CPY
            pltpu.sync_copy(agg.at[pl.ds(st, CPY)],
                            out_hbm.at[c, pl.ds(st, CPY)])

    return sc_kernel(x, col3, row3, val3)


def _mm_body(p0_ref, p1_ref, w_ref, b_ref, o_ref):
    acc = p0_ref[...] + p1_ref[...]
    o_ref[...] = lax.dot(acc, w_ref[...],
                         precision=lax.Precision.HIGHEST,
                         preferred_element_type=jnp.float32) + b_ref[...]


def _tc_matmul(p0, p1, W, b2):
    blk = 1280
    return pl.pallas_call(
        _mm_body,
        grid=(NPAD // blk,),
        in_specs=[
            pl.BlockSpec((blk, D), lambda i: (i, 0)),
            pl.BlockSpec((blk, D), lambda i: (i, 0)),
            pl.BlockSpec((D, D), lambda i: (0, 0)),
            pl.BlockSpec((1, D), lambda i: (0, 0)),
        ],
        out_specs=pl.BlockSpec((blk, D), lambda i: (i, 0)),
        out_shape=jax.ShapeDtypeStruct((NPAD, D), jnp.float32),
    )(p0, p1, W, b2)


def kernel(x, edge_index, edge_vals, W, b):
    row = edge_index[0]
    col = edge_index[1]
    e = row.shape[0]
    pad = EPAD - e
    spread = jnp.arange(pad, dtype=jnp.int32) % N
    row3 = jnp.concatenate([row, spread]).reshape(NT, NCH, CH)
    col3 = jnp.concatenate([col, spread]).reshape(NT, NCH, CH)
    val3 = jnp.concatenate(
        [edge_vals, jnp.zeros((pad,), jnp.float32)]).reshape(NT, NCH, CH)

    partials = _sc_segment_sum(x, col3, row3, val3)
    out = _tc_matmul(partials[0], partials[1], W, b.reshape(1, D))
    return out[:N]


# speedup vs baseline: 7.4363x; 1.3224x over previous
import dataclasses
import functools

import jax
import jax.numpy as jnp
from jax import lax
from jax.experimental import pallas as pl
from jax.experimental.pallas import tpu as pltpu
from jax.experimental.pallas import tpu_sc as plsc

N = 10000
D = 128
NC = 2
NS = 16
NT = NC * NS
CH = 128
NCH = 80
EPAD = NT * NCH * CH
NPAD = 10240
ROWS_PER_TILE = NPAD // NS
CPY = 128


def _sc_segment_sum(x, packed, valb):
    mesh = plsc.VectorSubcoreMesh(core_axis_name="c", subcore_axis_name="s")
    cp = pltpu.CompilerParams()
    if "needs_layout_passes" in pltpu.CompilerParams.__dataclass_fields__:
        cp = dataclasses.replace(cp, needs_layout_passes=False)

    @functools.partial(
        pl.kernel,
        compiler_params=cp,
        out_type=jax.ShapeDtypeStruct((NC, NPAD, D), jnp.float32),
        mesh=mesh,
        scratch_types=[
            pltpu.VMEM((NCH * CH,), jnp.int32),
            pltpu.VMEM((2, CH), jnp.int32),
            pltpu.VMEM((2, CH), jnp.int32),
            pltpu.VMEM((CH, D), jnp.float32),
            pltpu.VMEM((CH, D), jnp.float32),
            pltpu.VMEM((CH * 16,), jnp.float32),
            pltpu.VMEM((CH * 16,), jnp.float32),
            pltpu.VMEM_SHARED((NPAD, D), jnp.float32),
            pltpu.SemaphoreType.DMA,
            pltpu.SemaphoreType.DMA,
        ],
    )
    def sc_kernel(x_hbm, pk_hbm, valb_hbm, out_hbm,
                  pk_v, colb, rowb,
                  buf0, buf1, vb0, vb1, agg, sem0, sem1):
        c = lax.axis_index("c")
        s = lax.axis_index("s")
        wid = c * NS + s

        zero = jnp.zeros((16,), jnp.float32)

        @pl.loop(0, CH)
        def _zero_buf(r):
            for k in range(D // 16):
                buf0[r, pl.ds(k * 16, 16)] = zero

        for i in range(ROWS_PER_TILE // CPY):
            pltpu.sync_copy(buf0.at[pl.ds(0, CPY)],
                            agg.at[pl.ds(s * ROWS_PER_TILE + i * CPY, CPY)])

        pltpu.sync_copy(pk_hbm.at[wid], pk_v)

        def unpack(jj, p_):
            for g in range(CH // 16):
                p = pk_v[pl.ds(jj * CH + g * 16, 16)]
                colb[p_, pl.ds(g * 16, 16)] = p & 0x3FFF
                rowb[p_, pl.ds(g * 16, 16)] = lax.shift_right_logical(p, 14)

        def issue(jj, p_, bufp, vbp, semp):
            pltpu.async_copy(x_hbm.at[colb.at[p_]], bufp, semp)
            pltpu.async_copy(valb_hbm.at[wid, jj], vbp, semp)

        def consume(jj, p_, bufp, vbp, semp):
            pltpu.make_async_copy(x_hbm.at[colb.at[p_]], bufp, semp).wait()
            pltpu.make_async_copy(valb_hbm.at[wid, jj], vbp, semp).wait()

            @pl.loop(0, CH, step=4)
            def _scale(e0):
                for u in range(4):
                    e = e0 + u
                    v = vbp[pl.ds(e * 16, 16)]
                    for k in range(D // 16):
                        sl = pl.ds(k * 16, 16)
                        bufp[e, sl] = bufp[e, sl] * v

            pltpu.sync_copy(bufp, agg.at[rowb.at[p_]], add=True)

        unpack(0, 0)
        issue(0, 0, buf0, vb0, sem0)
        unpack(1, 1)
        issue(1, 1, buf1, vb1, sem1)

        @pl.loop(0, NCH - 2, step=2)
        def _chunk(j):
            consume(j, 0, buf0, vb0, sem0)
            unpack(j + 2, 0)
            issue(j + 2, 0, buf0, vb0, sem0)
            consume(j + 1, 1, buf1, vb1, sem1)
            unpack(j + 3, 1)
            issue(j + 3, 1, buf1, vb1, sem1)

        consume(NCH - 2, 0, buf0, vb0, sem0)
        consume(NCH - 1, 1, buf1, vb1, sem1)

        plsc.subcore_barrier()
        for i in range(ROWS_PER_TILE // CPY):
            st = s * ROWS_PER_TILE + i * CPY
            pltpu.sync_copy(agg.at[pl.ds(st, CPY)],
                            out_hbm.at[c, pl.ds(st, CPY)])

    return sc_kernel(x, packed, valb)


def _mm_body(p0_ref, p1_ref, w_ref, b_ref, o_ref):
    acc = p0_ref[...] + p1_ref[...]
    o_ref[...] = lax.dot(acc, w_ref[...],
                         precision=lax.Precision.HIGHEST,
                         preferred_element_type=jnp.float32) + b_ref[...]


def _tc_matmul(p0, p1, W, b2):
    blk = 1280
    return pl.pallas_call(
        _mm_body,
        grid=(NPAD // blk,),
        in_specs=[
            pl.BlockSpec((blk, D), lambda i: (i, 0)),
            pl.BlockSpec((blk, D), lambda i: (i, 0)),
            pl.BlockSpec((D, D), lambda i: (0, 0)),
            pl.BlockSpec((1, D), lambda i: (0, 0)),
        ],
        out_specs=pl.BlockSpec((blk, D), lambda i: (i, 0)),
        out_shape=jax.ShapeDtypeStruct((NPAD, D), jnp.float32),
    )(p0, p1, W, b2)


def kernel(x, edge_index, edge_vals, W, b):
    row = edge_index[0]
    col = edge_index[1]
    e = row.shape[0]
    pad = EPAD - e
    spread = jnp.arange(pad, dtype=jnp.int32) % N
    rowp = jnp.concatenate([row, spread])
    colp = jnp.concatenate([col, spread])
    packed = jnp.left_shift(rowp, 14) | colp
    packed = packed.reshape(NT, NCH * CH)
    valp = jnp.concatenate([edge_vals, jnp.zeros((pad,), jnp.float32)])
    valb = jnp.broadcast_to(
        valp[:, None], (EPAD, 16)).reshape(NT, NCH, CH * 16)

    partials = _sc_segment_sum(x, packed, valb)
    out = _tc_matmul(partials[0], partials[1], W, b.reshape(1, D))
    return out[:N]


# speedup vs baseline: 10.5878x; 1.4238x over previous
import dataclasses
import functools

import jax
import jax.numpy as jnp
from jax import lax
from jax.experimental import pallas as pl
from jax.experimental.pallas import tpu as pltpu
from jax.experimental.pallas import tpu_sc as plsc

N = 10000
D = 128
NC = 2
NS = 16
NT = NC * NS
CH = 64
NCH = 160
EPAD = NT * NCH * CH
NPAD = 10240
ROWS_PER_TILE = NPAD // NS
CPY = 128


def _sc_segment_sum(x, packed, valb):
    mesh = plsc.VectorSubcoreMesh(core_axis_name="c", subcore_axis_name="s")
    cp = pltpu.CompilerParams()
    if "needs_layout_passes" in pltpu.CompilerParams.__dataclass_fields__:
        cp = dataclasses.replace(cp, needs_layout_passes=False)

    @functools.partial(
        pl.kernel,
        compiler_params=cp,
        out_type=jax.ShapeDtypeStruct((NC, NPAD, D), jnp.float32),
        mesh=mesh,
        scratch_types=[
            pltpu.VMEM((NCH * CH,), jnp.int32),
            pltpu.VMEM((4, CH), jnp.int32),
            pltpu.VMEM((4, CH), jnp.int32),
            pltpu.VMEM((4, CH), jnp.float32),
            pltpu.VMEM((CH, D), jnp.float32),
            pltpu.VMEM((CH, D), jnp.float32),
            pltpu.VMEM((CH, D), jnp.float32),
            pltpu.VMEM((CH, D), jnp.float32),
            pltpu.VMEM_SHARED((NPAD, D), jnp.float32),
            pltpu.SemaphoreType.DMA,
            pltpu.SemaphoreType.DMA,
            pltpu.SemaphoreType.DMA,
            pltpu.SemaphoreType.DMA,
            pltpu.SemaphoreType.DMA,
            pltpu.SemaphoreType.DMA,
            pltpu.SemaphoreType.DMA,
            pltpu.SemaphoreType.DMA,
        ],
    )
    def sc_kernel(x_hbm, pk_hbm, val_hbm, out_hbm,
                  pk_v, colb, rowb, valc, b0, b1, b2, b3, agg,
                  g0, g1, g2, g3, s0, s1, s2, s3):
        c = lax.axis_index("c")
        s = lax.axis_index("s")
        wid = c * NS + s
        bufs = (b0, b1, b2, b3)
        gsems = (g0, g1, g2, g3)
        ssems = (s0, s1, s2, s3)

        zero = jnp.zeros((16,), jnp.float32)

        @pl.loop(0, CH)
        def _zero_buf(r):
            for k in range(D // 16):
                b0[r, pl.ds(k * 16, 16)] = zero

        for i in range(ROWS_PER_TILE // CPY):
            base = s * ROWS_PER_TILE + i * CPY
            pltpu.sync_copy(b0.at[pl.ds(0, CH)], agg.at[pl.ds(base, CH)])
            pltpu.sync_copy(b0.at[pl.ds(0, CH)], agg.at[pl.ds(base + CH, CH)])

        pltpu.sync_copy(pk_hbm.at[wid], pk_v)

        def unpack(jj, slot):
            for g in range(CH // 16):
                p = pk_v[pl.ds(jj * CH + g * 16, 16)]
                colb[slot, pl.ds(g * 16, 16)] = p & 0x3FFF
                rowb[slot, pl.ds(g * 16, 16)] = lax.shift_right_logical(p, 14)

        def issue(jj, slot):
            pltpu.async_copy(x_hbm.at[colb.at[slot]], bufs[slot], gsems[slot])
            pltpu.async_copy(val_hbm.at[wid, pl.ds(jj * CH, CH)],
                             valc.at[slot], gsems[slot])

        def wait_gather(jj, slot):
            pltpu.make_async_copy(
                x_hbm.at[colb.at[slot]], bufs[slot], gsems[slot]).wait()
            pltpu.make_async_copy(val_hbm.at[wid, pl.ds(jj * CH, CH)],
                                  valc.at[slot], gsems[slot]).wait()

        def scale(slot):
            bufp = bufs[slot]
            p16 = jnp.full((16,), slot, jnp.int32)

            @pl.loop(0, CH, step=16)
            def _scale(g):
                for t in range(16):
                    e = g + t
                    v = plsc.load_gather(
                        valc, [p16, jnp.full((16,), e, jnp.int32)])
                    for k in range(D // 16):
                        sl = pl.ds(k * 16, 16)
                        bufp[e, sl] = bufp[e, sl] * v

        def start_scatter(slot):
            pltpu.async_copy(bufs[slot], agg.at[rowb.at[slot]],
                             ssems[slot], add=True)

        def wait_scatter(slot):
            pltpu.make_async_copy(bufs[slot], agg.at[rowb.at[slot]],
                                  ssems[slot]).wait()

        def stage(jj, slot, first):
            nslot = (slot + 2) % 4
            if not first:
                wait_scatter(nslot)
            unpack(jj + 2, nslot)
            issue(jj + 2, nslot)
            wait_gather(jj, slot)
            scale(slot)
            start_scatter(slot)

        unpack(0, 0)
        issue(0, 0)
        unpack(1, 1)
        issue(1, 1)
        stage(0, 0, True)
        stage(1, 1, True)

        @pl.loop(2, NCH - 2, step=4)
        def _chunk(j):
            stage(j, 2, False)
            stage(j + 1, 3, False)
            stage(j + 2, 0, False)
            stage(j + 3, 1, False)

        wait_scatter(0)
        wait_gather(NCH - 2, 2)
        scale(2)
        start_scatter(2)
        wait_scatter(1)
        wait_gather(NCH - 1, 3)
        scale(3)
        start_scatter(3)
        wait_scatter(2)
        wait_scatter(3)

        plsc.subcore_barrier()
        for i in range(ROWS_PER_TILE // CPY):
            st = s * ROWS_PER_TILE + i * CPY
            pltpu.sync_copy(agg.at[pl.ds(st, CPY)],
                            out_hbm.at[c, pl.ds(st, CPY)])

    return sc_kernel(x, packed, valb)


def _mm_body(p0_ref, p1_ref, w_ref, b_ref, o_ref):
    acc = p0_ref[...] + p1_ref[...]
    o_ref[...] = lax.dot(acc, w_ref[...],
                         precision=lax.Precision.HIGHEST,
                         preferred_element_type=jnp.float32) + b_ref[...]


def _tc_matmul(p0, p1, W, b2):
    blk = 1000
    return pl.pallas_call(
        _mm_body,
        grid=(N // blk,),
        in_specs=[
            pl.BlockSpec((blk, D), lambda i: (i, 0)),
            pl.BlockSpec((blk, D), lambda i: (i, 0)),
            pl.BlockSpec((D, D), lambda i: (0, 0)),
            pl.BlockSpec((1, D), lambda i: (0, 0)),
        ],
        out_specs=pl.BlockSpec((blk, D), lambda i: (i, 0)),
        out_shape=jax.ShapeDtypeStruct((N, D), jnp.float32),
    )(p0, p1, W, b2)


def kernel(x, edge_index, edge_vals, W, b):
    row = edge_index[0]
    col = edge_index[1]
    e = row.shape[0]
    pad = EPAD - e
    spread = jnp.arange(pad, dtype=jnp.int32) % N
    rowp = jnp.concatenate([row, spread])
    colp = jnp.concatenate([col, spread])
    packed = jnp.left_shift(rowp, 14) | colp
    packed = packed.reshape(NT, NCH * CH)
    valp = jnp.concatenate([edge_vals, jnp.zeros((pad,), jnp.float32)])
    val2 = valp.reshape(NT, NCH * CH)

    partials = _sc_segment_sum(x, packed, val2)
    return _tc_matmul(partials[0], partials[1], W, b.reshape(1, D))


# speedup vs baseline: 10.7606x; 1.0163x over previous
import dataclasses
import functools

import numpy as np

import jax
import jax.numpy as jnp
from jax import lax
from jax.experimental import pallas as pl
from jax.experimental.pallas import tpu as pltpu
from jax.experimental.pallas import tpu_sc as plsc

N = 10000
D = 128
NC = 2
NS = 16
NT = NC * NS
CH = 64
NCH = 160
EPAD = NT * NCH * CH
NPAD = 10240
ROWS_PER_TILE = NPAD // NS
CPY = 128


def _sc_segment_sum(x, packed, valb):
    mesh = plsc.VectorSubcoreMesh(core_axis_name="c", subcore_axis_name="s")
    cp = pltpu.CompilerParams()
    if "needs_layout_passes" in pltpu.CompilerParams.__dataclass_fields__:
        cp = dataclasses.replace(cp, needs_layout_passes=False)

    @functools.partial(
        pl.kernel,
        compiler_params=cp,
        out_type=jax.ShapeDtypeStruct((NC, NPAD, D), jnp.float32),
        mesh=mesh,
        scratch_types=[
            pltpu.VMEM((NCH * CH,), jnp.int32),
            pltpu.VMEM((4, CH), jnp.int32),
            pltpu.VMEM((4, CH), jnp.int32),
            pltpu.VMEM((4, CH), jnp.float32),
            pltpu.VMEM((CH, D), jnp.float32),
            pltpu.VMEM((CH, D), jnp.float32),
            pltpu.VMEM((CH, D), jnp.float32),
            pltpu.VMEM((CH, D), jnp.float32),
            pltpu.VMEM_SHARED((NPAD, D), jnp.float32),
            pltpu.SemaphoreType.DMA,
            pltpu.SemaphoreType.DMA,
            pltpu.SemaphoreType.DMA,
            pltpu.SemaphoreType.DMA,
            pltpu.SemaphoreType.DMA,
            pltpu.SemaphoreType.DMA,
            pltpu.SemaphoreType.DMA,
            pltpu.SemaphoreType.DMA,
        ],
    )
    def sc_kernel(x_hbm, pk_hbm, val_hbm, out_hbm,
                  pk_v, colb, rowb, valc, b0, b1, b2, b3, agg,
                  g0, g1, g2, g3, s0, s1, s2, s3):
        c = lax.axis_index("c")
        s = lax.axis_index("s")
        wid = c * NS + s
        bufs = (b0, b1, b2, b3)
        gsems = (g0, g1, g2, g3)
        ssems = (s0, s1, s2, s3)

        zero = jnp.zeros((16,), jnp.float32)

        @pl.loop(0, CH)
        def _zero_buf(r):
            for k in range(D // 16):
                b0[r, pl.ds(k * 16, 16)] = zero

        for i in range(ROWS_PER_TILE // CPY):
            base = s * ROWS_PER_TILE + i * CPY
            pltpu.sync_copy(b0.at[pl.ds(0, CH)], agg.at[pl.ds(base, CH)])
            pltpu.sync_copy(b0.at[pl.ds(0, CH)], agg.at[pl.ds(base + CH, CH)])

        pltpu.sync_copy(pk_hbm.at[wid], pk_v)

        def unpack(jj, slot):
            for g in range(CH // 16):
                p = pk_v[pl.ds(jj * CH + g * 16, 16)]
                colb[slot, pl.ds(g * 16, 16)] = p & 0x3FFF
                rowb[slot, pl.ds(g * 16, 16)] = lax.shift_right_logical(p, 14)

        def issue(jj, slot):
            pltpu.async_copy(x_hbm.at[colb.at[slot]], bufs[slot], gsems[slot])
            pltpu.async_copy(val_hbm.at[wid, pl.ds(jj * CH, CH)],
                             valc.at[slot], gsems[slot])

        def wait_gather(jj, slot):
            pltpu.make_async_copy(
                x_hbm.at[colb.at[slot]], bufs[slot], gsems[slot]).wait()
            pltpu.make_async_copy(val_hbm.at[wid, pl.ds(jj * CH, CH)],
                                  valc.at[slot], gsems[slot]).wait()

        def scale(slot):
            bufp = bufs[slot]
            p16 = jnp.full((16,), slot, jnp.int32)

            @pl.loop(0, CH, step=16)
            def _scale(g):
                for t in range(16):
                    e = g + t
                    v = plsc.load_gather(
                        valc, [p16, jnp.full((16,), e, jnp.int32)])
                    for k in range(D // 16):
                        sl = pl.ds(k * 16, 16)
                        bufp[e, sl] = bufp[e, sl] * v

        def start_scatter(slot):
            pltpu.async_copy(bufs[slot], agg.at[rowb.at[slot]],
                             ssems[slot], add=True)

        def wait_scatter(slot):
            pltpu.make_async_copy(bufs[slot], agg.at[rowb.at[slot]],
                                  ssems[slot]).wait()

        def stage(jj, slot, first):
            nslot = (slot + 2) % 4
            if not first:
                wait_scatter(nslot)
            unpack(jj + 2, nslot)
            issue(jj + 2, nslot)
            wait_gather(jj, slot)
            scale(slot)
            start_scatter(slot)

        unpack(0, 0)
        issue(0, 0)
        unpack(1, 1)
        issue(1, 1)
        stage(0, 0, True)
        stage(1, 1, True)

        @pl.loop(2, NCH - 2, step=4)
        def _chunk(j):
            stage(j, 2, False)
            stage(j + 1, 3, False)
            stage(j + 2, 0, False)
            stage(j + 3, 1, False)

        wait_scatter(0)
        wait_gather(NCH - 2, 2)
        scale(2)
        start_scatter(2)
        wait_scatter(1)
        wait_gather(NCH - 1, 3)
        scale(3)
        start_scatter(3)
        wait_scatter(2)
        wait_scatter(3)

        plsc.subcore_barrier()
        for i in range(ROWS_PER_TILE // CPY):
            st = s * ROWS_PER_TILE + i * CPY
            pltpu.sync_copy(agg.at[pl.ds(st, CPY)],
                            out_hbm.at[c, pl.ds(st, CPY)])

    return sc_kernel(x, packed, valb)


def _mm_body(p0_ref, p1_ref, w_ref, b_ref, o_ref):
    acc = p0_ref[...] + p1_ref[...]
    o_ref[...] = lax.dot(acc, w_ref[...],
                         preferred_element_type=jnp.float32) + b_ref[...]


def _tc_matmul(p0, p1, W, b2):
    blk = 1000
    return pl.pallas_call(
        _mm_body,
        grid=(N // blk,),
        in_specs=[
            pl.BlockSpec((blk, D), lambda i: (i, 0)),
            pl.BlockSpec((blk, D), lambda i: (i, 0)),
            pl.BlockSpec((D, D), lambda i: (0, 0)),
            pl.BlockSpec((1, D), lambda i: (0, 0)),
        ],
        out_specs=pl.BlockSpec((blk, D), lambda i: (i, 0)),
        out_shape=jax.ShapeDtypeStruct((N, D), jnp.float32),
    )(p0, p1, W, b2)


def kernel(x, edge_index, edge_vals, W, b):
    row = edge_index[0]
    col = edge_index[1]
    e = row.shape[0]
    pad = EPAD - e
    spread_np = np.arange(pad, dtype=np.int32) % N
    pad_packed = jnp.asarray((spread_np << 14) | spread_np)
    packed = jnp.concatenate([jnp.left_shift(row, 14) | col, pad_packed])
    packed = packed.reshape(NT, NCH * CH)
    valp = jnp.concatenate([edge_vals, jnp.zeros((pad,), jnp.float32)])
    val2 = valp.reshape(NT, NCH * CH)

    partials = _sc_segment_sum(x, packed, val2)
    return _tc_matmul(partials[0], partials[1], W, b.reshape(1, D))
